# Initial kernel scaffold; baseline (speedup 1.0000x reference)
import jax
import jax.numpy as jnp
from jax.experimental import pallas as pl


def kernel(x, edge_index, batch, W1, b1, W2, b2, Wl, bl):
    raise NotImplementedError("write your pallas kernel here")


# speedup vs baseline: 11.9887x; 11.9887x over previous
import functools

import jax
import jax.numpy as jnp
from jax import lax
from jax.experimental import pallas as pl
from jax.experimental.pallas import tpu as pltpu
from jax.experimental.pallas import tpu_sc as plsc

NC = 2
NS = 16
NW = NC * NS

CHUNK = 80


def _sc_mesh():
    return plsc.VectorSubcoreMesh(core_axis_name="c", subcore_axis_name="s")


def _make_deg_kernel(E, N, DW=128):
    per_tile = E // NW
    nch = per_tile // CHUNK
    nz = 10
    rps = N // nz

    @functools.partial(
        pl.kernel,
        out_type=jax.ShapeDtypeStruct((NC, N, DW), jnp.float32),
        mesh=_sc_mesh(),
        scratch_types=[
            pltpu.VMEM((CHUNK,), jnp.int32),
            pltpu.VMEM((CHUNK, DW), jnp.float32),
            pltpu.VMEM_SHARED((N, DW), jnp.float32),
        ],
    )
    def deg_kernel(dst_hbm, ones_hbm, zeros_hbm, out_hbm, idx_v, ones_v, acc):
        c = lax.axis_index("c")
        s = lax.axis_index("s")
        wid = s * NC + c
        pltpu.sync_copy(ones_hbm, ones_v)

        @pl.when(s < nz)
        def _():
            pltpu.sync_copy(zeros_hbm.at[pl.ds(s * rps, rps)],
                            acc.at[pl.ds(s * rps, rps)])

        plsc.subcore_barrier()
        base = wid * per_tile

        def body(j, carry):
            pltpu.sync_copy(dst_hbm.at[pl.ds(base + j * CHUNK, CHUNK)], idx_v)
            pltpu.sync_copy(ones_v, acc.at[idx_v], add=True)
            return carry

        lax.fori_loop(0, nch, body, 0)
        plsc.subcore_barrier()

        @pl.when(s < nz)
        def _():
            pltpu.sync_copy(acc.at[pl.ds(s * rps, rps)],
                            out_hbm.at[c, pl.ds(s * rps, rps)])

    return deg_kernel


def _make_scatter_kernel(E, N, H):
    per_tile = E // NW
    nch = per_tile // CHUNK
    nz = 10
    rps = N // nz

    @functools.partial(
        pl.kernel,
        out_type=jax.ShapeDtypeStruct((NC, N, H), jnp.float32),
        mesh=_sc_mesh(),
        scratch_types=[
            pltpu.VMEM((CHUNK,), jnp.int32),
            pltpu.VMEM((CHUNK,), jnp.int32),
            pltpu.VMEM((CHUNK, H), jnp.float32),
            pltpu.VMEM_SHARED((N, H), jnp.float32),
            pltpu.SemaphoreType.DMA,
        ],
    )
    def scat_kernel(p_hbm, src_hbm, dst_hbm, zeros_hbm, out_hbm,
                    src_v, dst_v, rows_v, acc, sem):
        c = lax.axis_index("c")
        s = lax.axis_index("s")
        wid = s * NC + c

        @pl.when(s < nz)
        def _():
            pltpu.sync_copy(zeros_hbm.at[pl.ds(s * rps, rps)],
                            acc.at[pl.ds(s * rps, rps)])

        plsc.subcore_barrier()
        base = wid * per_tile

        def body(j, carry):
            pltpu.sync_copy(src_hbm.at[pl.ds(base + j * CHUNK, CHUNK)], src_v)
            pltpu.sync_copy(dst_hbm.at[pl.ds(base + j * CHUNK, CHUNK)], dst_v)
            pltpu.async_copy(p_hbm.at[src_v], rows_v, sem).wait()
            pltpu.sync_copy(rows_v, acc.at[dst_v], add=True)
            return carry

        lax.fori_loop(0, nch, body, 0)
        plsc.subcore_barrier()

        @pl.when(s < nz)
        def _():
            pltpu.sync_copy(acc.at[pl.ds(s * rps, rps)],
                            out_hbm.at[c, pl.ds(s * rps, rps)])

    return scat_kernel


def _tc_k1(x, W1, dega, degb, B):
    N, F = x.shape
    H = W1.shape[1]
    grid = N // B

    def body(x_ref, w_ref, da_ref, db_ref, p_ref, dinv_ref):
        deg = da_ref[...] + db_ref[...] + 1.0
        dinv = lax.rsqrt(deg)
        z = jnp.dot(x_ref[...], w_ref[...], preferred_element_type=jnp.float32)
        p_ref[...] = dinv * z
        dinv_ref[...] = dinv

    return pl.pallas_call(
        body,
        grid=(grid,),
        in_specs=[
            pl.BlockSpec((B, F), lambda i: (i, 0)),
            pl.BlockSpec((F, H), lambda i: (0, 0)),
            pl.BlockSpec((B, 1), lambda i: (i, 0)),
            pl.BlockSpec((B, 1), lambda i: (i, 0)),
        ],
        out_specs=[
            pl.BlockSpec((B, H), lambda i: (i, 0)),
            pl.BlockSpec((B, 1), lambda i: (i, 0)),
        ],
        out_shape=[
            jax.ShapeDtypeStruct((N, H), jnp.float32),
            jax.ShapeDtypeStruct((N, 1), jnp.float32),
        ],
    )(x, W1, dega, degb)


def _tc_k2(Sa, Sb, p1, dinv, b1, W2, B):
    N, H = p1.shape
    grid = N // B

    def body(sa_ref, sb_ref, p_ref, dinv_ref, b1_ref, w2_ref, p2_ref):
        agg = sa_ref[...] + sb_ref[...] + p_ref[...]
        h1 = jnp.maximum(dinv_ref[...] * agg + b1_ref[...], 0.0)
        z2 = jnp.dot(h1, w2_ref[...], preferred_element_type=jnp.float32)
        p2_ref[...] = dinv_ref[...] * z2

    return pl.pallas_call(
        body,
        grid=(grid,),
        in_specs=[
            pl.BlockSpec((B, H), lambda i: (i, 0)),
            pl.BlockSpec((B, H), lambda i: (i, 0)),
            pl.BlockSpec((B, H), lambda i: (i, 0)),
            pl.BlockSpec((B, 1), lambda i: (i, 0)),
            pl.BlockSpec((1, H), lambda i: (0, 0)),
            pl.BlockSpec((H, H), lambda i: (0, 0)),
        ],
        out_specs=pl.BlockSpec((B, H), lambda i: (i, 0)),
        out_shape=jax.ShapeDtypeStruct((N, H), jnp.float32),
    )(Sa, Sb, p1, dinv, b1, W2)


def _tc_k3(Sa, Sb, p2, dinv, b2, batch2, Wl, bl, G, B):
    N, H = p2.shape
    C = Wl.shape[1]
    grid = N // B

    def body(sa_ref, sb_ref, p_ref, dinv_ref, b2_ref, bat_ref, wl_ref, bl_ref,
             out_ref, pooled):
        i = pl.program_id(0)
        agg = sa_ref[...] + sb_ref[...] + p_ref[...]
        h2 = jnp.maximum(dinv_ref[...] * agg + b2_ref[...], 0.0)
        gids = lax.broadcasted_iota(jnp.int32, (B, G), 1)
        onehot = (bat_ref[...] == gids).astype(jnp.float32)
        blk = lax.dot_general(onehot, h2, (((0,), (0,)), ((), ())),
                              preferred_element_type=jnp.float32)

        @pl.when(i == 0)
        def _():
            pooled[...] = blk

        @pl.when(i > 0)
        def _():
            pooled[...] = pooled[...] + blk

        @pl.when(i == grid - 1)
        def _():
            out_ref[...] = jnp.dot(pooled[...], wl_ref[...],
                                   preferred_element_type=jnp.float32) + bl_ref[...]

    return pl.pallas_call(
        body,
        grid=(grid,),
        in_specs=[
            pl.BlockSpec((B, H), lambda i: (i, 0)),
            pl.BlockSpec((B, H), lambda i: (i, 0)),
            pl.BlockSpec((B, H), lambda i: (i, 0)),
            pl.BlockSpec((B, 1), lambda i: (i, 0)),
            pl.BlockSpec((1, H), lambda i: (0, 0)),
            pl.BlockSpec((B, 1), lambda i: (i, 0)),
            pl.BlockSpec((H, C), lambda i: (0, 0)),
            pl.BlockSpec((1, C), lambda i: (0, 0)),
        ],
        out_specs=pl.BlockSpec((G, C), lambda i: (0, 0)),
        out_shape=jax.ShapeDtypeStruct((G, C), jnp.float32),
        scratch_shapes=[pltpu.VMEM((G, H), jnp.float32)],
    )(Sa, Sb, p2, dinv, b2, batch2, Wl, bl)


@jax.jit
def kernel(x, edge_index, batch, W1, b1, W2, b2, Wl, bl):
    N, F = x.shape
    H = W1.shape[1]
    E = edge_index.shape[1]
    G = 64
    B = 1000

    src = edge_index[0]
    dst = edge_index[1]
    ones_c = jnp.ones((CHUNK, 128), jnp.float32)
    zeros1 = jnp.zeros((N, 128), jnp.float32)
    zerosH = jnp.zeros((N, H), jnp.float32)

    deg2 = _make_deg_kernel(E, N)(dst, ones_c, zeros1)
    dega, degb = deg2[0, :, 0:1], deg2[1, :, 0:1]

    p1, dinv = _tc_k1(x, W1, dega, degb, B)

    scat = _make_scatter_kernel(E, N, H)
    S1 = scat(p1, src, dst, zerosH)
    p2 = _tc_k2(S1[0], S1[1], p1, dinv, b1.reshape(1, H), W2, B)

    S2 = scat(p2, src, dst, zerosH)
    return _tc_k3(S2[0], S2[1], p2, dinv, b2.reshape(1, H),
                  batch.reshape(N, 1), Wl, bl.reshape(1, Wl.shape[1]), G, B)
